# baseline (speedup 1.0000x reference)
import functools

import jax
import jax.numpy as jnp
from jax import lax
from jax.experimental import pallas as pl
from jax.experimental.pallas import tpu as pltpu
from jax.experimental.pallas import tpu_sc as plsc

F32 = jnp.float32

_N = 100000
_E = 3200000
_NP = 100352
_NC = 2
_NS = 16
_NW = _NC * _NS
_EPW = _E // _NW
_K = 1000
_KP = 2000
_NWINP = _EPW // _KP
_NWIN = _EPW // _K
_CHN = _NP // _NS
_BLK = 2048
_BM = 4000
_BM2 = 2000
_KS = 800
_NWINS = _EPW // _KS

def _worker_id():
    return lax.axis_index("s") * _NC + lax.axis_index("c")


@functools.cache
def _sc_kernels():
    mesh = plsc.VectorSubcoreMesh(core_axis_name="c", subcore_axis_name="s")
    cparams = pltpu.CompilerParams(use_tc_tiling_on_sc=False)

    @functools.partial(
        pl.kernel,
        mesh=mesh,
        compiler_params=cparams,
        out_type=[
            jax.ShapeDtypeStruct((_E, 16), F32),
            jax.ShapeDtypeStruct((_E, 16), F32),
        ],
        scratch_types=[
            pltpu.VMEM((2, _K), jnp.int32),
            pltpu.VMEM((2, _K), jnp.int32),
            pltpu.VMEM((2, _K, 16), F32),
            pltpu.VMEM((2, _K, 16), F32),
            pltpu.SemaphoreType.DMA,
            pltpu.SemaphoreType.DMA,
            pltpu.SemaphoreType.DMA,
            pltpu.SemaphoreType.DMA,
            pltpu.SemaphoreType.DMA,
            pltpu.SemaphoreType.DMA,
        ],
    )
    def sc_gather(row_h, col_h, pr_h, pc_h, gr_o, gc_o, idxr, idxc, bufr,
                  bufc, smr0, smc0, smr1, smc1, smw0, smw1):
        base = _worker_id() * _EPW
        smr = [smr0, smr1]
        smc = [smc0, smc1]
        smw = [smw0, smw1]

        def issue(w, b):
            off = base + w * _K
            pltpu.sync_copy(row_h.at[pl.ds(off, _K)], idxr.at[b])
            pltpu.sync_copy(col_h.at[pl.ds(off, _K)], idxc.at[b])
            pltpu.async_copy(pr_h.at[idxr.at[b]], bufr.at[b], smr[b])
            pltpu.async_copy(pc_h.at[idxc.at[b]], bufc.at[b], smc[b])

        def finish(w, b):
            off = base + w * _K
            pltpu.make_async_copy(pr_h.at[idxr.at[b]], bufr.at[b],
                                  smr[b]).wait()
            pltpu.make_async_copy(pc_h.at[idxc.at[b]], bufc.at[b],
                                  smc[b]).wait()
            pltpu.async_copy(bufr.at[b], gr_o.at[pl.ds(off, _K)], smw[b])
            pltpu.async_copy(bufc.at[b], gc_o.at[pl.ds(off, _K)], smw[b])

        def drainw(b):
            pltpu.make_async_copy(bufr.at[b], gr_o.at[pl.ds(0, _K)],
                                  smw[b]).wait()
            pltpu.make_async_copy(bufc.at[b], gc_o.at[pl.ds(0, _K)],
                                  smw[b]).wait()

        issue(0, 0)
        issue(1, 1)

        def _win(i, _):
            finish(2 * i, 0)
            finish(2 * i + 1, 1)
            drainw(0)
            issue(2 * i + 2, 0)
            drainw(1)
            issue(2 * i + 3, 1)
            return 0
        lax.fori_loop(0, _NWIN // 2 - 1, _win, 0)
        finish(_NWIN - 2, 0)
        finish(_NWIN - 1, 1)
        drainw(0)
        drainw(1)

    @functools.partial(
        pl.kernel,
        mesh=mesh,
        compiler_params=cparams,
        out_type=[
            jax.ShapeDtypeStruct((_E, 16), F32),
            jax.ShapeDtypeStruct((_E, 16), F32),
            jax.ShapeDtypeStruct((_E, 8), F32),
            jax.ShapeDtypeStruct((_E, 8), F32),
            jax.ShapeDtypeStruct((_NC, _NP), F32),
        ],
        scratch_types=[
            pltpu.VMEM((2, _K), jnp.int32),
            pltpu.VMEM((2, _K), jnp.int32),
            pltpu.VMEM((2, _K, 16), F32),
            pltpu.VMEM((2, _K, 16), F32),
            pltpu.VMEM((2, _K, 8), F32),
            pltpu.VMEM((2, _K, 8), F32),
            pltpu.VMEM((1008,), F32),
            pltpu.VMEM_SHARED((_NP,), F32),
            pltpu.SemaphoreType.DMA,
            pltpu.SemaphoreType.DMA,
            pltpu.SemaphoreType.DMA,
            pltpu.SemaphoreType.DMA,
            pltpu.SemaphoreType.DMA,
            pltpu.SemaphoreType.DMA,
            pltpu.SemaphoreType.DMA,
            pltpu.SemaphoreType.DMA,
        ],
    )
    def sc_gather_pos(row_h, col_h, pr_h, pc_h, posp_h, zeros1_h,
                      gr_o, gc_o, posr_o, posc_o, cnt_o,
                      idxr, idxc, bufr, bufc, bpr, bpc, ones, cnt_sp,
                      smr0, smc0, smr1, smc1, smp0, smp1, smw0, smw1):
        cid = lax.axis_index("c")
        sid = lax.axis_index("s")
        base = _worker_id() * _EPW
        smr = [smr0, smr1]
        smc = [smc0, smc1]
        smp = [smp0, smp1]
        smw = [smw0, smw1]

        pltpu.sync_copy(zeros1_h.at[pl.ds(sid * _CHN, _CHN)],
                        cnt_sp.at[pl.ds(sid * _CHN, _CHN)])

        def _ones_body(j, _):
            ones[pl.ds(j * 16, 16)] = jnp.full((16,), 1.0, F32)
            return 0
        lax.fori_loop(0, 63, _ones_body, 0, unroll=8)
        plsc.subcore_barrier()

        def issue(w, b):
            off = base + w * _K
            pltpu.sync_copy(row_h.at[pl.ds(off, _K)], idxr.at[b])
            pltpu.sync_copy(col_h.at[pl.ds(off, _K)], idxc.at[b])
            pltpu.async_copy(pr_h.at[idxr.at[b]], bufr.at[b], smr[b])
            pltpu.async_copy(pc_h.at[idxc.at[b]], bufc.at[b], smc[b])
            pltpu.async_copy(posp_h.at[idxr.at[b]], bpr.at[b], smp[b])
            pltpu.async_copy(posp_h.at[idxc.at[b]], bpc.at[b], smp[b])

        def finish(w, b):
            off = base + w * _K
            pltpu.make_async_copy(pr_h.at[idxr.at[b]], bufr.at[b],
                                  smr[b]).wait()
            pltpu.make_async_copy(pc_h.at[idxc.at[b]], bufc.at[b],
                                  smc[b]).wait()
            pltpu.make_async_copy(posp_h.at[idxr.at[b]], bpr.at[b],
                                  smp[b]).wait()
            pltpu.make_async_copy(posp_h.at[idxc.at[b]], bpc.at[b],
                                  smp[b]).wait()
            pltpu.async_copy(bufr.at[b], gr_o.at[pl.ds(off, _K)], smw[b])
            pltpu.async_copy(bufc.at[b], gc_o.at[pl.ds(off, _K)], smw[b])
            pltpu.async_copy(bpr.at[b], posr_o.at[pl.ds(off, _K)], smw[b])
            pltpu.async_copy(bpc.at[b], posc_o.at[pl.ds(off, _K)], smw[b])
            pltpu.sync_copy(ones.at[pl.ds(0, _K)], cnt_sp.at[idxc.at[b]],
                            add=True)

        def drainw(b):
            pltpu.make_async_copy(bufr.at[b], gr_o.at[pl.ds(0, _K)],
                                  smw[b]).wait()
            pltpu.make_async_copy(bufc.at[b], gc_o.at[pl.ds(0, _K)],
                                  smw[b]).wait()
            pltpu.make_async_copy(bpr.at[b], posr_o.at[pl.ds(0, _K)],
                                  smw[b]).wait()
            pltpu.make_async_copy(bpc.at[b], posc_o.at[pl.ds(0, _K)],
                                  smw[b]).wait()

        issue(0, 0)
        issue(1, 1)

        def _win(i, _):
            finish(2 * i, 0)
            finish(2 * i + 1, 1)
            drainw(0)
            issue(2 * i + 2, 0)
            drainw(1)
            issue(2 * i + 3, 1)
            return 0
        lax.fori_loop(0, _NWIN // 2 - 1, _win, 0)
        finish(_NWIN - 2, 0)
        finish(_NWIN - 1, 1)
        drainw(0)
        drainw(1)

        plsc.subcore_barrier()
        pltpu.sync_copy(cnt_sp.at[pl.ds(sid * _CHN, _CHN)],
                        cnt_o.at[cid, pl.ds(sid * _CHN, _CHN)])

    @functools.partial(
        pl.kernel,
        mesh=mesh,
        compiler_params=cparams,
        out_type=jax.ShapeDtypeStruct((_NC, _NP, 16), F32),
        scratch_types=[
            pltpu.VMEM((2, _KS), jnp.int32),
            pltpu.VMEM((2, _KS, 16), F32),
            pltpu.VMEM_SHARED((_NP, 16), F32),
            pltpu.SemaphoreType.DMA,
            pltpu.SemaphoreType.DMA,
        ],
    )
    def sc_scatter(col_h, m_h, zeros16_h, agg_o, idxc, bufm, agg_sp,
                   sm0, sm1):
        cid = lax.axis_index("c")
        sid = lax.axis_index("s")
        base = _worker_id() * _EPW
        sm = [sm0, sm1]

        pltpu.sync_copy(zeros16_h.at[pl.ds(sid * _CHN, _CHN)],
                        agg_sp.at[pl.ds(sid * _CHN, _CHN)])
        plsc.subcore_barrier()

        def issue(w, b):
            off = base + w * _KS
            pltpu.sync_copy(col_h.at[pl.ds(off, _KS)], idxc.at[b])
            pltpu.async_copy(m_h.at[pl.ds(off, _KS)], bufm.at[b], sm[b])

        def finish(w, b):
            off = base + w * _KS
            pltpu.make_async_copy(m_h.at[pl.ds(off, _KS)], bufm.at[b],
                                  sm[b]).wait()
            pltpu.sync_copy(bufm.at[b], agg_sp.at[idxc.at[b]], add=True)

        issue(0, 0)
        issue(1, 1)

        def _win(i, _):
            finish(2 * i, 0)
            issue(2 * i + 2, 0)
            finish(2 * i + 1, 1)
            issue(2 * i + 3, 1)
            return 0
        lax.fori_loop(0, (_NWINS - 1) // 2 - 1, _win, 0)
        finish(_NWINS - 3, 0)
        issue(_NWINS - 1, 0)
        finish(_NWINS - 2, 1)
        finish(_NWINS - 1, 0)

        plsc.subcore_barrier()
        pltpu.sync_copy(agg_sp.at[pl.ds(sid * _CHN, _CHN)],
                        agg_o.at[cid, pl.ds(sid * _CHN, _CHN)])

    return sc_gather_pos, sc_gather, sc_scatter


def _full_spec(shape):
    return pl.BlockSpec(shape, lambda i: (0,) * len(shape))


def _row_spec(blk, minor):
    return pl.BlockSpec((blk, minor), lambda i: (i, 0))


def _tc_dense_pre_body(x_ref, ww_ref, s_ref, sb_ref, wr_ref, wc_ref, mb_ref,
                       mv_ref, h0_ref, pr_ref, pc_ref):
    x = x_ref[...]
    mv_ref[...] = jnp.dot(x, ww_ref[...], preferred_element_type=F32)
    h0 = jnp.dot(x, s_ref[...], preferred_element_type=F32) + sb_ref[...]
    h0_ref[...] = h0
    pr_ref[...] = jnp.dot(h0, wr_ref[...], preferred_element_type=F32)
    pc_ref[...] = jnp.dot(h0, wc_ref[...], preferred_element_type=F32) + mb_ref[...]


def _tc_dense_pre(x128, ww, s, sbrow, wrT, wcT, mbrow):
    nblk = _NP // _BLK
    return pl.pallas_call(
        _tc_dense_pre_body,
        grid=(nblk,),
        in_specs=[
            _row_spec(_BLK, 128),
            _full_spec((128, 128)),
            _full_spec((128, 16)),
            _full_spec((1, 16)),
            _full_spec((16, 16)),
            _full_spec((16, 16)),
            _full_spec((1, 16)),
        ],
        out_specs=[
            _row_spec(_BLK, 128),
            _row_spec(_BLK, 16),
            _row_spec(_BLK, 16),
            _row_spec(_BLK, 16),
        ],
        out_shape=[
            jax.ShapeDtypeStruct((_NP, 128), F32),
            jax.ShapeDtypeStruct((_NP, 16), F32),
            jax.ShapeDtypeStruct((_NP, 16), F32),
            jax.ShapeDtypeStruct((_NP, 16), F32),
        ],
    )(x128, ww, s, sbrow, wrT, wcT, mbrow)


def _tc_edgeattr_body(pr_ref, pc_ref, gsel_ref, d_ref):
    df = pr_ref[...] - pc_ref[...]
    d2 = jnp.dot(df * df, gsel_ref[...], preferred_element_type=F32)
    d_ref[...] = jnp.sqrt(d2)


def _tc_edgeattr(posr128, posc128, gsel):
    nblk = (_E // 16) // _BM2
    return pl.pallas_call(
        _tc_edgeattr_body,
        grid=(nblk,),
        in_specs=[
            _row_spec(_BM2, 128),
            _row_spec(_BM2, 128),
            _full_spec((128, 16)),
        ],
        out_specs=_row_spec(_BM2, 16),
        out_shape=jax.ShapeDtypeStruct((_E // 16, 16), F32),
    )(posr128, posc128, gsel)


def _tc_msg_body(gr_ref, gc_ref, d_ref, b8_ref, wt_ref, gt_ref, bt_ref,
                 m16_ref, m_ref):
    z = gr_ref[...] + gc_ref[...] + jnp.dot(d_ref[...], b8_ref[...],
                                            preferred_element_type=F32) * wt_ref[...]
    g = jax.nn.gelu(z)
    m16 = m16_ref[...]
    mu = jnp.dot(g, m16, preferred_element_type=F32)
    xc = g - mu
    var = jnp.dot(xc * xc, m16, preferred_element_type=F32)
    m_ref[...] = xc * lax.rsqrt(var + 1e-5) * gt_ref[...] + bt_ref[...]


def _tc_msg(gr128, gc128, d8, b8, wt, gt, bt, m16):
    nblk = (_E // 8) // _BM
    return pl.pallas_call(
        _tc_msg_body,
        grid=(nblk,),
        in_specs=[
            _row_spec(_BM, 128),
            _row_spec(_BM, 128),
            _row_spec(_BM, 8),
            _full_spec((8, 128)),
            _full_spec((1, 128)),
            _full_spec((1, 128)),
            _full_spec((1, 128)),
            _full_spec((128, 128)),
        ],
        out_specs=_row_spec(_BM, 128),
        out_shape=jax.ShapeDtypeStruct((_E // 8, 128), F32),
    )(gr128, gc128, d8, b8, wt, gt, bt, m16)


def _tc_upd_body(h_ref, p0_ref, p1_ref, c0_ref, c1_ref, uh_ref, ua_ref,
                 ub_ref, ug_ref, ube_ref, m16_ref, wr_ref, wc_ref, mb_ref,
                 hn_ref, pr_ref, pc_ref):
    h = h_ref[...]
    c = jnp.maximum(c0_ref[...] + c1_ref[...], 1.0)
    agg = (p0_ref[...] + p1_ref[...]) / c
    u = (jnp.dot(h, uh_ref[...], preferred_element_type=F32)
         + jnp.dot(agg, ua_ref[...], preferred_element_type=F32)
         + ub_ref[...])
    m16 = m16_ref[...]
    mu = jnp.dot(u, m16, preferred_element_type=F32)
    uc = u - mu
    var = jnp.dot(uc * uc, m16, preferred_element_type=F32)
    hn = h + uc * lax.rsqrt(var + 1e-5) * ug_ref[...] + ube_ref[...]
    hn_ref[...] = hn
    pr_ref[...] = jnp.dot(hn, wr_ref[...], preferred_element_type=F32)
    pc_ref[...] = jnp.dot(hn, wc_ref[...], preferred_element_type=F32) + mb_ref[...]


def _tc_upd(h, p0, p1, c0, c1, uhT, uaT, ubrow, ugrow, uberow, m16s,
            wrT, wcT, mbrow):
    nblk = _NP // _BLK
    return pl.pallas_call(
        _tc_upd_body,
        grid=(nblk,),
        in_specs=[
            _row_spec(_BLK, 16),
            _row_spec(_BLK, 16),
            _row_spec(_BLK, 16),
            _row_spec(_BLK, 1),
            _row_spec(_BLK, 1),
            _full_spec((16, 16)),
            _full_spec((16, 16)),
            _full_spec((1, 16)),
            _full_spec((1, 16)),
            _full_spec((1, 16)),
            _full_spec((16, 16)),
            _full_spec((16, 16)),
            _full_spec((16, 16)),
            _full_spec((1, 16)),
        ],
        out_specs=[
            _row_spec(_BLK, 16),
            _row_spec(_BLK, 16),
            _row_spec(_BLK, 16),
        ],
        out_shape=[
            jax.ShapeDtypeStruct((_NP, 16), F32),
            jax.ShapeDtypeStruct((_NP, 16), F32),
            jax.ShapeDtypeStruct((_NP, 16), F32),
        ],
    )(h, p0, p1, c0, c1, uhT, uaT, ubrow, ugrow, uberow, m16s, wrT, wcT, mbrow)


def _pack_equilinear(W0, W1, W2, W3, sW):
    wstack = jnp.stack([W0, W1, W1, W1, W2, W2, W2, W3])
    eye8 = jnp.eye(8, dtype=F32)
    ww = jnp.einsum('coi,cd->icod', wstack, eye8).reshape(128, 128)
    onehot0 = eye8[0]
    s = jnp.einsum('oi,c->ico', sW, onehot0).reshape(128, 16)
    return ww, s


def kernel(x_mv, x_s, cartesian_pos, edge_index, W0, W1, W2, W3, sW, sb,
           msg_W, msg_b, msg_g, msg_beta, upd_W, upd_b, upd_g, upd_beta):
    n = x_mv.shape[0]
    row = edge_index[0]
    col = edge_index[1]

    x128 = jnp.pad(x_mv.reshape(n, 128), ((0, _NP - n), (0, 0)))
    posp = jnp.pad(cartesian_pos, ((0, _NP - n), (0, 5)))
    zeros1 = jnp.zeros((_NP,), F32)
    zeros16 = jnp.zeros((_NP, 16), F32)
    ww, s = _pack_equilinear(W0, W1, W2, W3, sW)
    sbrow = sb.reshape(1, 16)

    wrT = [msg_W[t][:, :16].T for t in range(2)]
    wcT = [msg_W[t][:, 16:32].T for t in range(2)]
    we_t = [jnp.tile(msg_W[t][:, 32], 8).reshape(1, 128) for t in range(2)]
    mbrow = [msg_b[t].reshape(1, 16) for t in range(2)]
    gt = [jnp.tile(msg_g[t], 8).reshape(1, 128) for t in range(2)]
    bt = [jnp.tile(msg_beta[t], 8).reshape(1, 128) for t in range(2)]
    uhT = [upd_W[t][:, :16].T for t in range(2)]
    uaT = [upd_W[t][:, 16:].T for t in range(2)]
    ubrow = [upd_b[t].reshape(1, 16) for t in range(2)]
    ugrow = [upd_g[t].reshape(1, 16) for t in range(2)]
    uberow = [upd_beta[t].reshape(1, 16) for t in range(2)]

    lane = jnp.arange(128)
    m16 = ((lane[:, None] // 16) == (lane[None, :] // 16)).astype(F32) / 16.0
    m16s = jnp.full((16, 16), 1.0 / 16.0, F32)
    b8 = (jnp.arange(8)[:, None] == (lane[None, :] // 16)).astype(F32)
    gsel = (((lane[:, None] // 8) == jnp.arange(16)[None, :])
            & ((lane[:, None] % 8) < 3)).astype(F32)

    sc_gather_pos, sc_gather, sc_scatter = _sc_kernels()

    mv128, h, pr, pc = _tc_dense_pre(x128, ww, s, sbrow, wrT[0], wcT[0],
                                     mbrow[0])

    gr, gc, posr8, posc8, cnt = sc_gather_pos(row, col, pr, pc, posp, zeros1)
    c0 = cnt[0].reshape(_NP, 1)
    c1 = cnt[1].reshape(_NP, 1)

    d = _tc_edgeattr(posr8.reshape(_E // 16, 128),
                     posc8.reshape(_E // 16, 128), gsel)
    d8 = d.reshape(_E // 8, 8)

    for t in range(2):
        if t > 0:
            gr, gc = sc_gather(row, col, pr, pc)
        m = _tc_msg(gr.reshape(_E // 8, 128), gc.reshape(_E // 8, 128), d8,
                    b8, we_t[t], gt[t], bt[t], m16)
        agg = sc_scatter(col, m.reshape(_E, 16), zeros16)
        tn = min(t + 1, 1)
        h, pr, pc = _tc_upd(h, agg[0], agg[1], c0, c1, uhT[t], uaT[t],
                            ubrow[t], ugrow[t], uberow[t], m16s,
                            wrT[tn], wcT[tn], mbrow[tn])

    mv_out = mv128[:n].reshape(n, 16, 8)
    return (mv_out, h[:n])


# speedup vs baseline: 1.0514x; 1.0514x over previous
import functools

import jax
import jax.numpy as jnp
from jax import lax
from jax.experimental import pallas as pl
from jax.experimental.pallas import tpu as pltpu
from jax.experimental.pallas import tpu_sc as plsc

F32 = jnp.float32

_N = 100000
_E = 3200000
_NP = 100352
_NC = 2
_NS = 16
_NW = _NC * _NS
_EPW = _E // _NW
_K = 1000
_KP = 2000
_NWINP = _EPW // _KP
_NWIN = _EPW // _K
_CHN = _NP // _NS
_BLK = 2048
_BM = 4000
_BM2 = 2000
_KS = 800
_NWINS = _EPW // _KS

def _worker_id():
    return lax.axis_index("s") * _NC + lax.axis_index("c")


@functools.cache
def _sc_kernels():
    mesh = plsc.VectorSubcoreMesh(core_axis_name="c", subcore_axis_name="s")
    cparams = pltpu.CompilerParams(use_tc_tiling_on_sc=False)

    @functools.partial(
        pl.kernel,
        mesh=mesh,
        compiler_params=cparams,
        out_type=[
            jax.ShapeDtypeStruct((_E, 8), F32),
            jax.ShapeDtypeStruct((_E, 8), F32),
            jax.ShapeDtypeStruct((_NC, _NP), F32),
        ],
        scratch_types=[
            pltpu.VMEM((2, _KP), jnp.int32),
            pltpu.VMEM((2, _KP), jnp.int32),
            pltpu.VMEM((2, _KP, 8), F32),
            pltpu.VMEM((2, _KP, 8), F32),
            pltpu.VMEM((_KP,), F32),
            pltpu.VMEM_SHARED((_NP,), F32),
            pltpu.SemaphoreType.DMA,
            pltpu.SemaphoreType.DMA,
            pltpu.SemaphoreType.DMA,
            pltpu.SemaphoreType.DMA,
        ],
    )
    def sc_pos_counts(row_h, col_h, posp_h, zeros1_h, posr_o, posc_o, cnt_o,
                      idxr, idxc, bufr, bufc, ones, cnt_sp, smr0, smc0,
                      smr1, smc1):
        cid = lax.axis_index("c")
        sid = lax.axis_index("s")
        base = _worker_id() * _EPW
        smr = [smr0, smr1]
        smc = [smc0, smc1]

        pltpu.sync_copy(zeros1_h.at[pl.ds(sid * _CHN, _CHN)],
                        cnt_sp.at[pl.ds(sid * _CHN, _CHN)])

        def _ones_body(j, _):
            ones[pl.ds(j * 16, 16)] = jnp.full((16,), 1.0, F32)
            return 0
        lax.fori_loop(0, _KP // 16, _ones_body, 0, unroll=8)
        plsc.subcore_barrier()

        def issue(w, b):
            off = base + w * _KP
            pltpu.sync_copy(row_h.at[pl.ds(off, _KP)], idxr.at[b])
            pltpu.sync_copy(col_h.at[pl.ds(off, _KP)], idxc.at[b])
            pltpu.async_copy(posp_h.at[idxr.at[b]], bufr.at[b], smr[b])
            pltpu.async_copy(posp_h.at[idxc.at[b]], bufc.at[b], smc[b])

        def finish(w, b):
            off = base + w * _KP
            pltpu.make_async_copy(posp_h.at[idxr.at[b]], bufr.at[b],
                                  smr[b]).wait()
            pltpu.make_async_copy(posp_h.at[idxc.at[b]], bufc.at[b],
                                  smc[b]).wait()
            pltpu.async_copy(bufr.at[b], posr_o.at[pl.ds(off, _KP)], smr[b])
            pltpu.async_copy(bufc.at[b], posc_o.at[pl.ds(off, _KP)], smc[b])
            pltpu.sync_copy(ones, cnt_sp.at[idxc.at[b]], add=True)

        def drainw(b):
            pltpu.make_async_copy(bufr.at[b], posr_o.at[pl.ds(0, _KP)],
                                  smr[b]).wait()
            pltpu.make_async_copy(bufc.at[b], posc_o.at[pl.ds(0, _KP)],
                                  smc[b]).wait()

        issue(0, 0)
        issue(1, 1)

        def _win(i, _):
            finish(2 * i, 0)
            finish(2 * i + 1, 1)
            drainw(0)
            issue(2 * i + 2, 0)
            drainw(1)
            issue(2 * i + 3, 1)
            return 0
        lax.fori_loop(0, _NWINP // 2 - 1, _win, 0)
        finish(_NWINP - 2, 0)
        finish(_NWINP - 1, 1)
        drainw(0)
        drainw(1)

        plsc.subcore_barrier()
        pltpu.sync_copy(cnt_sp.at[pl.ds(sid * _CHN, _CHN)],
                        cnt_o.at[cid, pl.ds(sid * _CHN, _CHN)])

    @functools.partial(
        pl.kernel,
        mesh=mesh,
        compiler_params=cparams,
        out_type=[
            jax.ShapeDtypeStruct((_E, 16), F32),
            jax.ShapeDtypeStruct((_E, 16), F32),
        ],
        scratch_types=[
            pltpu.VMEM((2, _K), jnp.int32),
            pltpu.VMEM((2, _K), jnp.int32),
            pltpu.VMEM((2, _K, 16), F32),
            pltpu.VMEM((2, _K, 16), F32),
            pltpu.SemaphoreType.DMA,
            pltpu.SemaphoreType.DMA,
            pltpu.SemaphoreType.DMA,
            pltpu.SemaphoreType.DMA,
            pltpu.SemaphoreType.DMA,
            pltpu.SemaphoreType.DMA,
        ],
    )
    def sc_gather(row_h, col_h, pr_h, pc_h, gr_o, gc_o, idxr, idxc, bufr,
                  bufc, smr0, smc0, smr1, smc1, smw0, smw1):
        base = _worker_id() * _EPW
        smr = [smr0, smr1]
        smc = [smc0, smc1]
        smw = [smw0, smw1]

        def issue(w, b):
            off = base + w * _K
            pltpu.sync_copy(row_h.at[pl.ds(off, _K)], idxr.at[b])
            pltpu.sync_copy(col_h.at[pl.ds(off, _K)], idxc.at[b])
            pltpu.async_copy(pr_h.at[idxr.at[b]], bufr.at[b], smr[b])
            pltpu.async_copy(pc_h.at[idxc.at[b]], bufc.at[b], smc[b])

        def finish(w, b):
            off = base + w * _K
            pltpu.make_async_copy(pr_h.at[idxr.at[b]], bufr.at[b],
                                  smr[b]).wait()
            pltpu.make_async_copy(pc_h.at[idxc.at[b]], bufc.at[b],
                                  smc[b]).wait()
            pltpu.async_copy(bufr.at[b], gr_o.at[pl.ds(off, _K)], smw[b])
            pltpu.async_copy(bufc.at[b], gc_o.at[pl.ds(off, _K)], smw[b])

        def drainw(b):
            pltpu.make_async_copy(bufr.at[b], gr_o.at[pl.ds(0, _K)],
                                  smw[b]).wait()
            pltpu.make_async_copy(bufc.at[b], gc_o.at[pl.ds(0, _K)],
                                  smw[b]).wait()

        issue(0, 0)
        issue(1, 1)

        def _win(i, _):
            finish(2 * i, 0)
            finish(2 * i + 1, 1)
            drainw(0)
            issue(2 * i + 2, 0)
            drainw(1)
            issue(2 * i + 3, 1)
            return 0
        lax.fori_loop(0, _NWIN // 2 - 1, _win, 0)
        finish(_NWIN - 2, 0)
        finish(_NWIN - 1, 1)
        drainw(0)
        drainw(1)

    @functools.partial(
        pl.kernel,
        mesh=mesh,
        compiler_params=cparams,
        out_type=jax.ShapeDtypeStruct((_NC, _NP, 16), F32),
        scratch_types=[
            pltpu.VMEM((2, _KS), jnp.int32),
            pltpu.VMEM((2, _KS, 16), F32),
            pltpu.VMEM_SHARED((_NP, 16), F32),
            pltpu.SemaphoreType.DMA,
            pltpu.SemaphoreType.DMA,
        ],
    )
    def sc_scatter(col_h, m_h, zeros16_h, agg_o, idxc, bufm, agg_sp,
                   sm0, sm1):
        cid = lax.axis_index("c")
        sid = lax.axis_index("s")
        base = _worker_id() * _EPW
        sm = [sm0, sm1]

        pltpu.sync_copy(zeros16_h.at[pl.ds(sid * _CHN, _CHN)],
                        agg_sp.at[pl.ds(sid * _CHN, _CHN)])
        plsc.subcore_barrier()

        def issue(w, b):
            off = base + w * _KS
            pltpu.sync_copy(col_h.at[pl.ds(off, _KS)], idxc.at[b])
            pltpu.async_copy(m_h.at[pl.ds(off, _KS)], bufm.at[b], sm[b])

        def finish(w, b):
            off = base + w * _KS
            pltpu.make_async_copy(m_h.at[pl.ds(off, _KS)], bufm.at[b],
                                  sm[b]).wait()
            pltpu.sync_copy(bufm.at[b], agg_sp.at[idxc.at[b]], add=True)

        issue(0, 0)
        issue(1, 1)

        def _win(i, _):
            finish(2 * i, 0)
            issue(2 * i + 2, 0)
            finish(2 * i + 1, 1)
            issue(2 * i + 3, 1)
            return 0
        lax.fori_loop(0, (_NWINS - 1) // 2 - 1, _win, 0)
        finish(_NWINS - 3, 0)
        issue(_NWINS - 1, 0)
        finish(_NWINS - 2, 1)
        finish(_NWINS - 1, 0)

        plsc.subcore_barrier()
        pltpu.sync_copy(agg_sp.at[pl.ds(sid * _CHN, _CHN)],
                        agg_o.at[cid, pl.ds(sid * _CHN, _CHN)])

    return sc_pos_counts, sc_gather, sc_scatter


def _full_spec(shape):
    return pl.BlockSpec(shape, lambda i: (0,) * len(shape))


def _row_spec(blk, minor):
    return pl.BlockSpec((blk, minor), lambda i: (i, 0))


def _tc_dense_pre_body(x_ref, ww_ref, s_ref, sb_ref, wr_ref, wc_ref, mb_ref,
                       mv_ref, h0_ref, pr_ref, pc_ref):
    x = x_ref[...]
    mv_ref[...] = jnp.dot(x, ww_ref[...], preferred_element_type=F32)
    h0 = jnp.dot(x, s_ref[...], preferred_element_type=F32) + sb_ref[...]
    h0_ref[...] = h0
    pr_ref[...] = jnp.dot(h0, wr_ref[...], preferred_element_type=F32)
    pc_ref[...] = jnp.dot(h0, wc_ref[...], preferred_element_type=F32) + mb_ref[...]


def _tc_dense_pre(x128, ww, s, sbrow, wrT, wcT, mbrow):
    nblk = _NP // _BLK
    return pl.pallas_call(
        _tc_dense_pre_body,
        grid=(nblk,),
        in_specs=[
            _row_spec(_BLK, 128),
            _full_spec((128, 128)),
            _full_spec((128, 16)),
            _full_spec((1, 16)),
            _full_spec((16, 16)),
            _full_spec((16, 16)),
            _full_spec((1, 16)),
        ],
        out_specs=[
            _row_spec(_BLK, 128),
            _row_spec(_BLK, 16),
            _row_spec(_BLK, 16),
            _row_spec(_BLK, 16),
        ],
        out_shape=[
            jax.ShapeDtypeStruct((_NP, 128), F32),
            jax.ShapeDtypeStruct((_NP, 16), F32),
            jax.ShapeDtypeStruct((_NP, 16), F32),
            jax.ShapeDtypeStruct((_NP, 16), F32),
        ],
    )(x128, ww, s, sbrow, wrT, wcT, mbrow)


def _tc_edgeattr_body(pr_ref, pc_ref, gsel_ref, d_ref):
    df = pr_ref[...] - pc_ref[...]
    d2 = jnp.dot(df * df, gsel_ref[...], preferred_element_type=F32)
    d_ref[...] = jnp.sqrt(d2)


def _tc_edgeattr(posr128, posc128, gsel):
    nblk = (_E // 16) // _BM2
    return pl.pallas_call(
        _tc_edgeattr_body,
        grid=(nblk,),
        in_specs=[
            _row_spec(_BM2, 128),
            _row_spec(_BM2, 128),
            _full_spec((128, 16)),
        ],
        out_specs=_row_spec(_BM2, 16),
        out_shape=jax.ShapeDtypeStruct((_E // 16, 16), F32),
    )(posr128, posc128, gsel)


def _tc_msg_body(gr_ref, gc_ref, d_ref, b8_ref, wt_ref, gt_ref, bt_ref,
                 m16_ref, m_ref):
    z = gr_ref[...] + gc_ref[...] + jnp.dot(d_ref[...], b8_ref[...],
                                            preferred_element_type=F32) * wt_ref[...]
    g = jax.nn.gelu(z)
    m16 = m16_ref[...]
    mu = jnp.dot(g, m16, preferred_element_type=F32)
    xc = g - mu
    var = jnp.dot(xc * xc, m16, preferred_element_type=F32)
    m_ref[...] = xc * lax.rsqrt(var + 1e-5) * gt_ref[...] + bt_ref[...]


def _tc_msg(gr128, gc128, d8, b8, wt, gt, bt, m16):
    nblk = (_E // 8) // _BM
    return pl.pallas_call(
        _tc_msg_body,
        grid=(nblk,),
        in_specs=[
            _row_spec(_BM, 128),
            _row_spec(_BM, 128),
            _row_spec(_BM, 8),
            _full_spec((8, 128)),
            _full_spec((1, 128)),
            _full_spec((1, 128)),
            _full_spec((1, 128)),
            _full_spec((128, 128)),
        ],
        out_specs=_row_spec(_BM, 128),
        out_shape=jax.ShapeDtypeStruct((_E // 8, 128), F32),
    )(gr128, gc128, d8, b8, wt, gt, bt, m16)


def _tc_upd_body(h_ref, p0_ref, p1_ref, c0_ref, c1_ref, uh_ref, ua_ref,
                 ub_ref, ug_ref, ube_ref, m16_ref, wr_ref, wc_ref, mb_ref,
                 hn_ref, pr_ref, pc_ref):
    h = h_ref[...]
    c = jnp.maximum(c0_ref[...] + c1_ref[...], 1.0)
    agg = (p0_ref[...] + p1_ref[...]) / c
    u = (jnp.dot(h, uh_ref[...], preferred_element_type=F32)
         + jnp.dot(agg, ua_ref[...], preferred_element_type=F32)
         + ub_ref[...])
    m16 = m16_ref[...]
    mu = jnp.dot(u, m16, preferred_element_type=F32)
    uc = u - mu
    var = jnp.dot(uc * uc, m16, preferred_element_type=F32)
    hn = h + uc * lax.rsqrt(var + 1e-5) * ug_ref[...] + ube_ref[...]
    hn_ref[...] = hn
    pr_ref[...] = jnp.dot(hn, wr_ref[...], preferred_element_type=F32)
    pc_ref[...] = jnp.dot(hn, wc_ref[...], preferred_element_type=F32) + mb_ref[...]


def _tc_upd(h, p0, p1, c0, c1, uhT, uaT, ubrow, ugrow, uberow, m16s,
            wrT, wcT, mbrow):
    nblk = _NP // _BLK
    return pl.pallas_call(
        _tc_upd_body,
        grid=(nblk,),
        in_specs=[
            _row_spec(_BLK, 16),
            _row_spec(_BLK, 16),
            _row_spec(_BLK, 16),
            _row_spec(_BLK, 1),
            _row_spec(_BLK, 1),
            _full_spec((16, 16)),
            _full_spec((16, 16)),
            _full_spec((1, 16)),
            _full_spec((1, 16)),
            _full_spec((1, 16)),
            _full_spec((16, 16)),
            _full_spec((16, 16)),
            _full_spec((16, 16)),
            _full_spec((1, 16)),
        ],
        out_specs=[
            _row_spec(_BLK, 16),
            _row_spec(_BLK, 16),
            _row_spec(_BLK, 16),
        ],
        out_shape=[
            jax.ShapeDtypeStruct((_NP, 16), F32),
            jax.ShapeDtypeStruct((_NP, 16), F32),
            jax.ShapeDtypeStruct((_NP, 16), F32),
        ],
    )(h, p0, p1, c0, c1, uhT, uaT, ubrow, ugrow, uberow, m16s, wrT, wcT, mbrow)


def _pack_equilinear(W0, W1, W2, W3, sW):
    wstack = jnp.stack([W0, W1, W1, W1, W2, W2, W2, W3])
    eye8 = jnp.eye(8, dtype=F32)
    ww = jnp.einsum('coi,cd->icod', wstack, eye8).reshape(128, 128)
    onehot0 = eye8[0]
    s = jnp.einsum('oi,c->ico', sW, onehot0).reshape(128, 16)
    return ww, s


def kernel(x_mv, x_s, cartesian_pos, edge_index, W0, W1, W2, W3, sW, sb,
           msg_W, msg_b, msg_g, msg_beta, upd_W, upd_b, upd_g, upd_beta):
    n = x_mv.shape[0]
    row = edge_index[0]
    col = edge_index[1]

    x128 = jnp.pad(x_mv.reshape(n, 128), ((0, _NP - n), (0, 0)))
    posp = jnp.pad(cartesian_pos, ((0, _NP - n), (0, 5)))
    zeros1 = jnp.zeros((_NP,), F32)
    zeros16 = jnp.zeros((_NP, 16), F32)
    ww, s = _pack_equilinear(W0, W1, W2, W3, sW)
    sbrow = sb.reshape(1, 16)

    wrT = [msg_W[t][:, :16].T for t in range(2)]
    wcT = [msg_W[t][:, 16:32].T for t in range(2)]
    we_t = [jnp.tile(msg_W[t][:, 32], 8).reshape(1, 128) for t in range(2)]
    mbrow = [msg_b[t].reshape(1, 16) for t in range(2)]
    gt = [jnp.tile(msg_g[t], 8).reshape(1, 128) for t in range(2)]
    bt = [jnp.tile(msg_beta[t], 8).reshape(1, 128) for t in range(2)]
    uhT = [upd_W[t][:, :16].T for t in range(2)]
    uaT = [upd_W[t][:, 16:].T for t in range(2)]
    ubrow = [upd_b[t].reshape(1, 16) for t in range(2)]
    ugrow = [upd_g[t].reshape(1, 16) for t in range(2)]
    uberow = [upd_beta[t].reshape(1, 16) for t in range(2)]

    lane = jnp.arange(128)
    m16 = ((lane[:, None] // 16) == (lane[None, :] // 16)).astype(F32) / 16.0
    m16s = jnp.full((16, 16), 1.0 / 16.0, F32)
    b8 = (jnp.arange(8)[:, None] == (lane[None, :] // 16)).astype(F32)
    gsel = (((lane[:, None] // 8) == jnp.arange(16)[None, :])
            & ((lane[:, None] % 8) < 3)).astype(F32)

    sc_pos_counts, sc_gather, sc_scatter = _sc_kernels()

    mv128, h, pr, pc = _tc_dense_pre(x128, ww, s, sbrow, wrT[0], wcT[0],
                                     mbrow[0])

    posr8, posc8, cnt = sc_pos_counts(row, col, posp, zeros1)
    c0 = cnt[0].reshape(_NP, 1)
    c1 = cnt[1].reshape(_NP, 1)

    d = _tc_edgeattr(posr8.reshape(_E // 16, 128),
                     posc8.reshape(_E // 16, 128), gsel)
    d8 = d.reshape(_E // 8, 8)

    for t in range(2):
        gr, gc = sc_gather(row, col, pr, pc)
        m = _tc_msg(gr.reshape(_E // 8, 128), gc.reshape(_E // 8, 128), d8,
                    b8, we_t[t], gt[t], bt[t], m16)
        agg = sc_scatter(col, m.reshape(_E, 16), zeros16)
        tn = min(t + 1, 1)
        h, pr, pc = _tc_upd(h, agg[0], agg[1], c0, c1, uhT[t], uaT[t],
                            ubrow[t], ugrow[t], uberow[t], m16s,
                            wrT[tn], wcT[tn], mbrow[tn])

    mv_out = mv128[:n].reshape(n, 16, 8)
    return (mv_out, h[:n])
